# baseline (device time: 235501 ns/iter reference)
import jax
import jax.numpy as jnp
from jax import lax
from jax.experimental import pallas as pl
from jax.experimental.pallas import tpu as pltpu

N_DEV = 8
E_PER = 2
F_TILES = 2


def kernel(x, assign, W1, W2):
    t, d = x.shape
    e_per, _, f = W1.shape
    assert e_per == E_PER
    pad = 128
    dc = d + pad
    th = t // 2
    fw = f // F_TILES

    x_bf = x.astype(jnp.bfloat16)
    a_col = assign.astype(jnp.bfloat16).reshape(t, 1)
    a_pad = jnp.zeros((t, pad - 1), jnp.bfloat16)
    xcat = jnp.concatenate([x_bf, a_col, a_pad], axis=1)
    w1_cat = jnp.transpose(W1.astype(jnp.bfloat16), (1, 0, 2)).reshape(
        d, E_PER * f)
    w2_cat = W2.astype(jnp.bfloat16).reshape(E_PER * f, d)

    def body(x_ref, w1_ref, w2_ref, out_ref,
             xs_r, xs_l, csend_r, csend_l, crecv_r, crecv_l,
             ag_s_r, ag_r_r, ag_s_l, ag_r_l,
             rs_s_r, rs_r_r, rs_s_l, rs_r_l,
             credit_r, credit_l):
        my = lax.axis_index("i")
        left = (my + N_DEV - 1) % N_DEV
        right = (my + 1) % N_DEV

        rings = (
            (xs_r, csend_r, crecv_r, ag_s_r, ag_r_r, rs_s_r, rs_r_r,
             credit_r, right, left),
            (xs_l, csend_l, crecv_l, ag_s_l, ag_r_l, rs_s_l, rs_r_l,
             credit_l, left, right),
        )

        def mk_ag(ring, h):
            xs, _, _, ag_s, ag_r, _, _, _, dst, _ = ring
            return pltpu.make_async_remote_copy(
                src_ref=xs.at[h], dst_ref=xs.at[h + 1],
                send_sem=ag_s.at[h], recv_sem=ag_r.at[h],
                device_id=(dst,), device_id_type=pl.DeviceIdType.MESH,
            )

        def mk_rs(ring, k):
            _, csend, crecv, _, _, rs_s, rs_r, _, dst, _ = ring
            return pltpu.make_async_remote_copy(
                src_ref=csend.at[k % 2], dst_ref=crecv.at[k % 2],
                send_sem=rs_s.at[k], recv_sem=rs_r.at[k],
                device_id=(dst,), device_id_type=pl.DeviceIdType.MESH,
            )

        barrier = pltpu.get_barrier_semaphore()
        for nbr in (left, right):
            pl.semaphore_signal(
                barrier, inc=1,
                device_id=(nbr,), device_id_type=pl.DeviceIdType.MESH,
            )
        pl.semaphore_wait(barrier, 2)

        xs_r[0] = x_ref[0:th, :]
        xs_l[0] = x_ref[th:t, :]

        for ring in rings:
            mk_ag(ring, 0).start()

        n_tiles = E_PER * F_TILES
        tw = E_PER * f // n_tiles

        def partial_rows(xs, slot):
            chunk = xs[slot]
            xc = chunk[:, 0:d]
            a = chunk[:, d:d + 1]
            y = jnp.zeros((th, d), jnp.float32)
            for ft in range(n_tiles):
                e_val = (my * E_PER + ft // F_TILES).astype(jnp.bfloat16)
                c0 = ft * tw
                h1f = jnp.dot(xc, w1_ref[:, c0:c0 + tw],
                              preferred_element_type=jnp.float32)
                h1 = jnp.where(
                    jnp.logical_and(a == e_val, h1f > 0), h1f, 0.0,
                ).astype(jnp.bfloat16)
                y = y + jnp.dot(h1, w2_ref[c0:c0 + tw, :],
                                preferred_element_type=jnp.float32)
            return y

        out_ref[0:th, :] = partial_rows(xs_r, 0)
        out_ref[th:t, :] = partial_rows(xs_l, 0)

        def hop(k, carry):
            kn = jnp.minimum(k + 1, N_DEV - 2)
            kp2 = jnp.maximum(k - 2, 0)
            kp1 = jnp.maximum(k - 1, 0)

            for ring in rings:
                mk_ag(ring, k).wait_recv()

            @pl.when(k < N_DEV - 2)
            def _():
                for ring in rings:
                    mk_ag(ring, kn).start()

            @pl.when(k >= 2)
            def _():
                for ring in rings:
                    mk_rs(ring, kp2).wait_send()

            for ring in rings:
                xs, csend = ring[0], ring[1]
                csend[k % 2] = partial_rows(xs, k + 1).astype(jnp.bfloat16)

            @pl.when(k >= 1)
            def _():
                for ring in rings:
                    csend, crecv = ring[1], ring[2]
                    mk_rs(ring, kp1).wait_recv()
                    csend[k % 2] = csend[k % 2] + crecv[kp1 % 2]

            @pl.when(jnp.logical_and(k >= 1, k <= N_DEV - 3))
            def _():
                for ring in rings:
                    pl.semaphore_signal(
                        ring[7], inc=1,
                        device_id=(ring[9],),
                        device_id_type=pl.DeviceIdType.MESH,
                    )

            @pl.when(k >= 2)
            def _():
                for ring in rings:
                    pl.semaphore_wait(ring[7], 1)

            for ring in rings:
                mk_rs(ring, k).start()
            return carry

        lax.fori_loop(0, N_DEV - 1, hop, 0)

        for ring, r0 in ((rings[0], 0), (rings[1], th)):
            crecv = ring[2]
            mk_rs(ring, N_DEV - 2).wait_recv()
            out_ref[r0:r0 + th, :] = (
                out_ref[r0:r0 + th, :] + crecv[0].astype(jnp.float32))

        def drain(h, carry):
            for ring in rings:
                mk_ag(ring, h).wait_send()
            return carry

        lax.fori_loop(0, N_DEV - 1, drain, 0)
        for ring in rings:
            mk_rs(ring, N_DEV - 3).wait_send()
            mk_rs(ring, N_DEV - 2).wait_send()

    return pl.pallas_call(
        body,
        out_shape=jax.ShapeDtypeStruct((t, d), jnp.float32),
        in_specs=[
            pl.BlockSpec(memory_space=pltpu.VMEM),
            pl.BlockSpec(memory_space=pltpu.VMEM),
            pl.BlockSpec(memory_space=pltpu.VMEM),
        ],
        out_specs=pl.BlockSpec(memory_space=pltpu.VMEM),
        scratch_shapes=[
            pltpu.VMEM((N_DEV, th, dc), jnp.bfloat16),
            pltpu.VMEM((N_DEV, th, dc), jnp.bfloat16),
            pltpu.VMEM((2, th, d), jnp.bfloat16),
            pltpu.VMEM((2, th, d), jnp.bfloat16),
            pltpu.VMEM((2, th, d), jnp.bfloat16),
            pltpu.VMEM((2, th, d), jnp.bfloat16),
            pltpu.SemaphoreType.DMA((N_DEV - 1,)),
            pltpu.SemaphoreType.DMA((N_DEV - 1,)),
            pltpu.SemaphoreType.DMA((N_DEV - 1,)),
            pltpu.SemaphoreType.DMA((N_DEV - 1,)),
            pltpu.SemaphoreType.DMA((N_DEV - 1,)),
            pltpu.SemaphoreType.DMA((N_DEV - 1,)),
            pltpu.SemaphoreType.DMA((N_DEV - 1,)),
            pltpu.SemaphoreType.DMA((N_DEV - 1,)),
            pltpu.SemaphoreType.REGULAR,
            pltpu.SemaphoreType.REGULAR,
        ],
        compiler_params=pltpu.CompilerParams(
            collective_id=0,
            vmem_limit_bytes=62 * 1024 * 1024,
        ),
    )(xcat, w1_cat, w2_cat)


# device time: 144565 ns/iter; 1.6290x vs baseline; 1.6290x over previous
import jax
import jax.numpy as jnp
from jax import lax
from jax.experimental import pallas as pl
from jax.experimental.pallas import tpu as pltpu

N_DEV = 8
E_PER = 2
N_EXP = N_DEV * E_PER
C_E = 96
F_TILES = 2


def kernel(x, assign, W1, W2):
    t, d = x.shape
    e_per, _, f = W1.shape
    assert e_per == E_PER
    fw = f // F_TILES
    blk = E_PER * C_E

    x_bf = x.astype(jnp.bfloat16)
    w1_bf = W1.astype(jnp.bfloat16)
    w2_bf = W2.astype(jnp.bfloat16)

    order = jnp.argsort(assign)
    sorted_e = assign[order]
    starts = jnp.searchsorted(sorted_e, jnp.arange(N_EXP))
    rank = jnp.arange(t, dtype=jnp.int32) - starts[sorted_e]
    slot = sorted_e * C_E + rank
    xpad = jnp.zeros((N_EXP * C_E, d), jnp.bfloat16).at[slot].set(x_bf[order])
    xpad = xpad.reshape(N_DEV, blk, d)
    inv_rows = jnp.zeros((t,), jnp.int32).at[order].set(slot)

    def body(xsend, w1_ref, w2_ref, out_ref, xrecv, ys,
             ssem1, rsem1, ssem2, rsem2):
        my = lax.axis_index("i")

        def mk(src_buf, dst_buf, ssems, rsems, sidx, ridx, dev):
            return pltpu.make_async_remote_copy(
                src_ref=src_buf.at[sidx], dst_ref=dst_buf.at[ridx],
                send_sem=ssems.at[sidx], recv_sem=rsems.at[ridx],
                device_id=(dev,), device_id_type=pl.DeviceIdType.MESH,
            )

        barrier = pltpu.get_barrier_semaphore()

        def bar_sig(p, c):
            pl.semaphore_signal(
                barrier, inc=1,
                device_id=((my + p) % N_DEV,),
                device_id_type=pl.DeviceIdType.MESH,
            )
            return c

        lax.fori_loop(1, N_DEV, bar_sig, 0)
        pl.semaphore_wait(barrier, N_DEV - 1)

        def send1(p, c):
            peer = (my + p) % N_DEV
            mk(xsend, xrecv, ssem1, rsem1, peer, my, peer).start()
            return c

        lax.fori_loop(1, N_DEV, send1, 0)
        xrecv[my] = xsend[my]

        def recv1(p, c):
            src = (my + N_DEV - p) % N_DEV
            mk(xsend, xrecv, ssem1, rsem1, src, src, src).wait_recv()
            return c

        lax.fori_loop(1, N_DEV, recv1, 0)

        for le in range(E_PER):
            xle = xrecv[:, le * C_E:(le + 1) * C_E, :].reshape(
                N_DEV * C_E, d)
            y = jnp.zeros((N_DEV * C_E, d), jnp.float32)
            for ft in range(F_TILES):
                f0 = ft * fw
                h1 = jnp.dot(xle, w1_ref[le, :, f0:f0 + fw],
                             preferred_element_type=jnp.float32)
                h1 = jnp.maximum(h1, 0.0).astype(jnp.bfloat16)
                y = y + jnp.dot(h1, w2_ref[le, f0:f0 + fw, :],
                                preferred_element_type=jnp.float32)
            ys[:, le * C_E:(le + 1) * C_E, :] = (
                y.astype(jnp.bfloat16).reshape(N_DEV, C_E, d))

        def send2(p, c):
            peer = (my + p) % N_DEV
            mk(ys, out_ref, ssem2, rsem2, peer, my, peer).start()
            return c

        lax.fori_loop(1, N_DEV, send2, 0)
        out_ref[my] = ys[my]

        def recv2(p, c):
            src = (my + N_DEV - p) % N_DEV
            mk(ys, out_ref, ssem2, rsem2, src, src, src).wait_recv()
            return c

        lax.fori_loop(1, N_DEV, recv2, 0)

        def drain(p, c):
            peer = (my + p) % N_DEV
            mk(xsend, xrecv, ssem1, rsem1, peer, my, peer).wait_send()
            mk(ys, out_ref, ssem2, rsem2, peer, my, peer).wait_send()
            return c

        lax.fori_loop(1, N_DEV, drain, 0)

    ypad = pl.pallas_call(
        body,
        out_shape=jax.ShapeDtypeStruct((N_DEV, blk, d), jnp.bfloat16),
        in_specs=[
            pl.BlockSpec(memory_space=pltpu.VMEM),
            pl.BlockSpec(memory_space=pltpu.VMEM),
            pl.BlockSpec(memory_space=pltpu.VMEM),
        ],
        out_specs=pl.BlockSpec(memory_space=pltpu.VMEM),
        scratch_shapes=[
            pltpu.VMEM((N_DEV, blk, d), jnp.bfloat16),
            pltpu.VMEM((N_DEV, blk, d), jnp.bfloat16),
            pltpu.SemaphoreType.DMA((N_DEV,)),
            pltpu.SemaphoreType.DMA((N_DEV,)),
            pltpu.SemaphoreType.DMA((N_DEV,)),
            pltpu.SemaphoreType.DMA((N_DEV,)),
        ],
        compiler_params=pltpu.CompilerParams(
            collective_id=0,
            vmem_limit_bytes=62 * 1024 * 1024,
        ),
    )(xpad, w1_bf, w2_bf)

    yflat = ypad.reshape(N_EXP * C_E, d)
    return yflat[inv_rows].astype(jnp.float32)


# device time: 122573 ns/iter; 1.9213x vs baseline; 1.1794x over previous
import jax
import jax.numpy as jnp
from jax import lax
from jax.experimental import pallas as pl
from jax.experimental.pallas import tpu as pltpu

N_DEV = 8
E_PER = 2
N_EXP = N_DEV * E_PER
C_E = 96
F_TILES = 2


def kernel(x, assign, W1, W2):
    t, d = x.shape
    e_per, _, f = W1.shape
    assert e_per == E_PER
    fw = f // F_TILES
    blk = E_PER * C_E
    rows = N_EXP * C_E

    x_bf = x.astype(jnp.bfloat16)
    w1_bf = W1.astype(jnp.bfloat16)
    w2_bf = W2.astype(jnp.bfloat16)

    onehot = (assign[:, None] == jnp.arange(N_EXP)[None, :]).astype(jnp.int32)
    rank_full = jnp.cumsum(onehot, axis=0) - onehot
    rank = jnp.take_along_axis(rank_full, assign[:, None], axis=1)[:, 0]
    slot = assign * C_E + rank
    slot_row = slot.reshape(1, t)
    slot_col = slot.reshape(t, 1)

    def body(x_ref, sr_ref, sc_ref, w1_ref, w2_ref, out_ref,
             xsend, xrecv, ys, yrecv,
             ssem1, rsem1, ssem2, rsem2):
        my = lax.axis_index("i")

        def mk(src_buf, dst_buf, ssems, rsems, sidx, ridx, dev):
            return pltpu.make_async_remote_copy(
                src_ref=src_buf.at[sidx], dst_ref=dst_buf.at[ridx],
                send_sem=ssems.at[sidx], recv_sem=rsems.at[ridx],
                device_id=(dev,), device_id_type=pl.DeviceIdType.MESH,
            )

        barrier = pltpu.get_barrier_semaphore()

        def bar_sig(p, c):
            pl.semaphore_signal(
                barrier, inc=1,
                device_id=((my + p) % N_DEV,),
                device_id_type=pl.DeviceIdType.MESH,
            )
            return c

        lax.fori_loop(1, N_DEV, bar_sig, 0)
        pl.semaphore_wait(barrier, N_DEV - 1)

        perm = (lax.broadcasted_iota(jnp.int32, (rows, t), 0)
                == sr_ref[...]).astype(jnp.bfloat16)
        xpad = jnp.dot(perm, x_ref[...],
                       preferred_element_type=jnp.float32)
        xsend[...] = xpad.astype(jnp.bfloat16).reshape(N_DEV, blk, d)

        def send1(p, c):
            peer = (my + p) % N_DEV
            mk(xsend, xrecv, ssem1, rsem1, peer, my, peer).start()
            return c

        lax.fori_loop(1, N_DEV, send1, 0)
        xrecv[my] = xsend[my]

        def recv1(p, c):
            src = (my + N_DEV - p) % N_DEV
            mk(xsend, xrecv, ssem1, rsem1, src, src, src).wait_recv()
            return c

        lax.fori_loop(1, N_DEV, recv1, 0)

        for le in range(E_PER):
            xle = xrecv[:, le * C_E:(le + 1) * C_E, :].reshape(
                N_DEV * C_E, d)
            y = jnp.zeros((N_DEV * C_E, d), jnp.float32)
            for ft in range(F_TILES):
                f0 = ft * fw
                h1 = jnp.dot(xle, w1_ref[le, :, f0:f0 + fw],
                             preferred_element_type=jnp.float32)
                h1 = jnp.maximum(h1, 0.0).astype(jnp.bfloat16)
                y = y + jnp.dot(h1, w2_ref[le, f0:f0 + fw, :],
                                preferred_element_type=jnp.float32)
            ys[:, le * C_E:(le + 1) * C_E, :] = (
                y.astype(jnp.bfloat16).reshape(N_DEV, C_E, d))

        def send2(p, c):
            peer = (my + p) % N_DEV
            mk(ys, yrecv, ssem2, rsem2, peer, my, peer).start()
            return c

        lax.fori_loop(1, N_DEV, send2, 0)
        yrecv[my] = ys[my]

        def recv2(p, c):
            src = (my + N_DEV - p) % N_DEV
            mk(ys, yrecv, ssem2, rsem2, src, src, src).wait_recv()
            return c

        lax.fori_loop(1, N_DEV, recv2, 0)

        permt = (lax.broadcasted_iota(jnp.int32, (t, rows), 1)
                 == sc_ref[...]).astype(jnp.bfloat16)
        yflat = yrecv[...].reshape(rows, d)
        out_ref[...] = jnp.dot(permt, yflat,
                               preferred_element_type=jnp.float32)

        def drain(p, c):
            peer = (my + p) % N_DEV
            mk(xsend, xrecv, ssem1, rsem1, peer, my, peer).wait_send()
            mk(ys, yrecv, ssem2, rsem2, peer, my, peer).wait_send()
            return c

        lax.fori_loop(1, N_DEV, drain, 0)

    return pl.pallas_call(
        body,
        out_shape=jax.ShapeDtypeStruct((t, d), jnp.float32),
        in_specs=[pl.BlockSpec(memory_space=pltpu.VMEM)] * 5,
        out_specs=pl.BlockSpec(memory_space=pltpu.VMEM),
        scratch_shapes=[
            pltpu.VMEM((N_DEV, blk, d), jnp.bfloat16),
            pltpu.VMEM((N_DEV, blk, d), jnp.bfloat16),
            pltpu.VMEM((N_DEV, blk, d), jnp.bfloat16),
            pltpu.VMEM((N_DEV, blk, d), jnp.bfloat16),
            pltpu.SemaphoreType.DMA((N_DEV,)),
            pltpu.SemaphoreType.DMA((N_DEV,)),
            pltpu.SemaphoreType.DMA((N_DEV,)),
            pltpu.SemaphoreType.DMA((N_DEV,)),
        ],
        compiler_params=pltpu.CompilerParams(
            collective_id=0,
            vmem_limit_bytes=62 * 1024 * 1024,
        ),
    )(x_bf, slot_row, slot_col, w1_bf, w2_bf)


# device time: 112983 ns/iter; 2.0844x vs baseline; 1.0849x over previous
import jax
import jax.numpy as jnp
from jax import lax
from jax.experimental import pallas as pl
from jax.experimental.pallas import tpu as pltpu

N_DEV = 8
E_PER = 2
N_EXP = N_DEV * E_PER
C_E = 96
F_TILES = 2


def kernel(x, assign, W1, W2):
    t, d = x.shape
    e_per, _, f = W1.shape
    assert e_per == E_PER
    fw = f // F_TILES
    blk = E_PER * C_E
    rows = N_EXP * C_E

    x_bf = x.astype(jnp.bfloat16)
    w1_bf = W1.astype(jnp.bfloat16)
    w2_bf = W2.astype(jnp.bfloat16)

    onehot = (assign[:, None] == jnp.arange(N_EXP)[None, :]).astype(jnp.int32)
    rank_full = jnp.cumsum(onehot, axis=0) - onehot
    rank = jnp.sum(rank_full * onehot, axis=1)
    slot = assign * C_E + rank
    slot_row = slot.reshape(1, t)
    slot_col = slot.reshape(t, 1)

    def body(x_ref, sr_ref, sc_ref, w1_ref, w2_ref, out_ref,
             xsend, xrecv, ys, yrecv,
             ssem1, rsem1, ssem2, rsem2):
        my = lax.axis_index("i")

        def mk(src_buf, dst_buf, ssems, rsems, sidx, ridx, dev):
            return pltpu.make_async_remote_copy(
                src_ref=src_buf.at[sidx], dst_ref=dst_buf.at[ridx],
                send_sem=ssems.at[sidx], recv_sem=rsems.at[ridx],
                device_id=(dev,), device_id_type=pl.DeviceIdType.MESH,
            )

        barrier = pltpu.get_barrier_semaphore()

        def bar_sig(p, c):
            pl.semaphore_signal(
                barrier, inc=1,
                device_id=((my + p) % N_DEV,),
                device_id_type=pl.DeviceIdType.MESH,
            )
            return c

        lax.fori_loop(1, N_DEV, bar_sig, 0)
        pl.semaphore_wait(barrier, N_DEV - 1)

        perm = (lax.broadcasted_iota(jnp.int32, (rows, t), 0)
                == sr_ref[...]).astype(jnp.bfloat16)
        xpad = jnp.dot(perm, x_ref[...],
                       preferred_element_type=jnp.float32)
        xsend[...] = xpad.astype(jnp.bfloat16).reshape(N_DEV, blk, d)

        def send1(p, c):
            peer = (my + p) % N_DEV
            mk(xsend, xrecv, ssem1, rsem1, peer, my, peer).start()
            return c

        lax.fori_loop(1, N_DEV, send1, 0)
        xrecv[my] = xsend[my]

        def recv1(p, c):
            src = (my + N_DEV - p) % N_DEV
            mk(xsend, xrecv, ssem1, rsem1, src, src, src).wait_recv()
            return c

        lax.fori_loop(1, N_DEV, recv1, 0)

        for le in range(E_PER):
            xle = xrecv[:, le * C_E:(le + 1) * C_E, :].reshape(
                N_DEV * C_E, d)
            y = jnp.zeros((N_DEV * C_E, d), jnp.float32)
            for ft in range(F_TILES):
                f0 = ft * fw
                h1 = jnp.dot(xle, w1_ref[le, :, f0:f0 + fw],
                             preferred_element_type=jnp.float32)
                h1 = jnp.maximum(h1, 0.0).astype(jnp.bfloat16)
                y = y + jnp.dot(h1, w2_ref[le, f0:f0 + fw, :],
                                preferred_element_type=jnp.float32)
            ys[:, le * C_E:(le + 1) * C_E, :] = (
                y.astype(jnp.bfloat16).reshape(N_DEV, C_E, d))

        def send2(p, c):
            peer = (my + p) % N_DEV
            mk(ys, yrecv, ssem2, rsem2, peer, my, peer).start()
            return c

        lax.fori_loop(1, N_DEV, send2, 0)
        yrecv[my] = ys[my]

        def recv2(p, c):
            src = (my + N_DEV - p) % N_DEV
            mk(ys, yrecv, ssem2, rsem2, src, src, src).wait_recv()
            return c

        lax.fori_loop(1, N_DEV, recv2, 0)

        permt = (lax.broadcasted_iota(jnp.int32, (t, rows), 1)
                 == sc_ref[...]).astype(jnp.bfloat16)
        yflat = yrecv[...].reshape(rows, d)
        out_ref[...] = jnp.dot(permt, yflat,
                               preferred_element_type=jnp.float32)

        def drain(p, c):
            peer = (my + p) % N_DEV
            mk(xsend, xrecv, ssem1, rsem1, peer, my, peer).wait_send()
            mk(ys, yrecv, ssem2, rsem2, peer, my, peer).wait_send()
            return c

        lax.fori_loop(1, N_DEV, drain, 0)

    return pl.pallas_call(
        body,
        out_shape=jax.ShapeDtypeStruct((t, d), jnp.float32),
        in_specs=[pl.BlockSpec(memory_space=pltpu.VMEM)] * 5,
        out_specs=pl.BlockSpec(memory_space=pltpu.VMEM),
        scratch_shapes=[
            pltpu.VMEM((N_DEV, blk, d), jnp.bfloat16),
            pltpu.VMEM((N_DEV, blk, d), jnp.bfloat16),
            pltpu.VMEM((N_DEV, blk, d), jnp.bfloat16),
            pltpu.VMEM((N_DEV, blk, d), jnp.bfloat16),
            pltpu.SemaphoreType.DMA((N_DEV,)),
            pltpu.SemaphoreType.DMA((N_DEV,)),
            pltpu.SemaphoreType.DMA((N_DEV,)),
            pltpu.SemaphoreType.DMA((N_DEV,)),
        ],
        compiler_params=pltpu.CompilerParams(
            collective_id=0,
            vmem_limit_bytes=62 * 1024 * 1024,
        ),
    )(x_bf, slot_row, slot_col, w1_bf, w2_bf)


# device time: 103164 ns/iter; 2.2828x vs baseline; 1.0952x over previous
import jax
import jax.numpy as jnp
from jax import lax
from jax.experimental import pallas as pl
from jax.experimental.pallas import tpu as pltpu

N_DEV = 8
E_PER = 2
N_EXP = N_DEV * E_PER
C_E = 96
F_TILES = 2


def kernel(x, assign, W1, W2):
    t, d = x.shape
    e_per, _, f = W1.shape
    assert e_per == E_PER
    fw = f // F_TILES
    blk = E_PER * C_E
    rows = N_EXP * C_E

    x_bf = x.astype(jnp.bfloat16)

    onehot = (assign[:, None] == jnp.arange(N_EXP)[None, :]).astype(jnp.int32)
    rank_full = jnp.cumsum(onehot, axis=0) - onehot
    rank = jnp.sum(rank_full * onehot, axis=1)
    slot = assign * C_E + rank
    slot_row = slot.reshape(1, t)
    slot_col = slot.reshape(t, 1)

    def body(x_ref, sr_ref, sc_ref, w1_ref, w2_ref, out_ref,
             xsend, xrecv, yrecv,
             ssem1, rsem1, ssem2, rsem2):
        ys = xsend
        my = lax.axis_index("i")

        def mk(src_buf, dst_buf, ssems, rsems, sidx, ridx, dev):
            return pltpu.make_async_remote_copy(
                src_ref=src_buf.at[sidx], dst_ref=dst_buf.at[ridx],
                send_sem=ssems.at[sidx], recv_sem=rsems.at[ridx],
                device_id=(dev,), device_id_type=pl.DeviceIdType.MESH,
            )

        barrier = pltpu.get_barrier_semaphore()

        def bar_sig(p, c):
            pl.semaphore_signal(
                barrier, inc=1,
                device_id=((my + p) % N_DEV,),
                device_id_type=pl.DeviceIdType.MESH,
            )
            return c

        lax.fori_loop(1, N_DEV, bar_sig, 0)
        pl.semaphore_wait(barrier, N_DEV - 1)

        perm = (lax.broadcasted_iota(jnp.int32, (rows, t), 0)
                == sr_ref[...]).astype(jnp.bfloat16)
        xpad = jnp.dot(perm, x_ref[...],
                       preferred_element_type=jnp.float32)
        xsend[...] = xpad.astype(jnp.bfloat16).reshape(N_DEV, blk, d)

        def send1(p, c):
            peer = (my + p) % N_DEV
            mk(xsend, xrecv, ssem1, rsem1, peer, my, peer).start()
            return c

        lax.fori_loop(1, N_DEV, send1, 0)
        xrecv[my] = xsend[my]

        def recv1(p, c):
            src = (my + N_DEV - p) % N_DEV
            mk(xsend, xrecv, ssem1, rsem1, src, src, src).wait_recv()
            return c

        lax.fori_loop(1, N_DEV, recv1, 0)

        def drain1(p, c):
            peer = (my + p) % N_DEV
            mk(xsend, xrecv, ssem1, rsem1, peer, my, peer).wait_send()
            return c

        lax.fori_loop(1, N_DEV, drain1, 0)

        for le in range(E_PER):
            xle = xrecv[:, le * C_E:(le + 1) * C_E, :].reshape(
                N_DEV * C_E, d)
            y = jnp.zeros((N_DEV * C_E, d), jnp.float32)
            for ft in range(F_TILES):
                f0 = ft * fw
                w1t = w1_ref[le, :, f0:f0 + fw].astype(jnp.bfloat16)
                w2t = w2_ref[le, f0:f0 + fw, :].astype(jnp.bfloat16)
                h1 = jnp.dot(xle, w1t,
                             preferred_element_type=jnp.float32)
                h1 = jnp.maximum(h1, 0.0).astype(jnp.bfloat16)
                y = y + jnp.dot(h1, w2t,
                                preferred_element_type=jnp.float32)
            ys[:, le * C_E:(le + 1) * C_E, :] = (
                y.astype(jnp.bfloat16).reshape(N_DEV, C_E, d))

        def send2(p, c):
            peer = (my + p) % N_DEV
            mk(ys, yrecv, ssem2, rsem2, peer, my, peer).start()
            return c

        lax.fori_loop(1, N_DEV, send2, 0)
        yrecv[my] = ys[my]

        def recv2(p, c):
            src = (my + N_DEV - p) % N_DEV
            mk(ys, yrecv, ssem2, rsem2, src, src, src).wait_recv()
            return c

        lax.fori_loop(1, N_DEV, recv2, 0)

        permt = (lax.broadcasted_iota(jnp.int32, (t, rows), 1)
                 == sc_ref[...]).astype(jnp.bfloat16)
        yflat = yrecv[...].reshape(rows, d)
        out_ref[...] = jnp.dot(permt, yflat,
                               preferred_element_type=jnp.float32)

        def drain2(p, c):
            peer = (my + p) % N_DEV
            mk(ys, yrecv, ssem2, rsem2, peer, my, peer).wait_send()
            return c

        lax.fori_loop(1, N_DEV, drain2, 0)

    return pl.pallas_call(
        body,
        out_shape=jax.ShapeDtypeStruct((t, d), jnp.float32),
        in_specs=[pl.BlockSpec(memory_space=pltpu.VMEM)] * 5,
        out_specs=pl.BlockSpec(memory_space=pltpu.VMEM),
        scratch_shapes=[
            pltpu.VMEM((N_DEV, blk, d), jnp.bfloat16),
            pltpu.VMEM((N_DEV, blk, d), jnp.bfloat16),
            pltpu.VMEM((N_DEV, blk, d), jnp.bfloat16),
            pltpu.SemaphoreType.DMA((N_DEV,)),
            pltpu.SemaphoreType.DMA((N_DEV,)),
            pltpu.SemaphoreType.DMA((N_DEV,)),
            pltpu.SemaphoreType.DMA((N_DEV,)),
        ],
        compiler_params=pltpu.CompilerParams(
            collective_id=0,
            vmem_limit_bytes=62 * 1024 * 1024,
        ),
    )(x_bf, slot_row, slot_col, W1, W2)
